# baseline (speedup 1.0000x reference)
import jax
import jax.numpy as jnp
from jax.experimental import pallas as pl
from jax.experimental.pallas import tpu as pltpu

BATCH = 1024
STACK = 100000
FEAT = 128
KEEP = STACK - BATCH


def _shift_kernel(x_ref, stack_ref, out_ref, sem_keep, sem_new):
    keep = pltpu.make_async_copy(
        stack_ref.at[pl.ds(BATCH, KEEP), :],
        out_ref.at[pl.ds(0, KEEP), :],
        sem_keep,
    )
    new = pltpu.make_async_copy(
        x_ref,
        out_ref.at[pl.ds(KEEP, BATCH), :],
        sem_new,
    )
    keep.start()
    new.start()
    keep.wait()
    new.wait()


def kernel(x, latent_stack):
    return pl.pallas_call(
        _shift_kernel,
        out_shape=jax.ShapeDtypeStruct((STACK, FEAT), jnp.float32),
        in_specs=[
            pl.BlockSpec(memory_space=pl.ANY),
            pl.BlockSpec(memory_space=pl.ANY),
        ],
        out_specs=pl.BlockSpec(memory_space=pl.ANY),
        scratch_shapes=[pltpu.SemaphoreType.DMA, pltpu.SemaphoreType.DMA],
    )(x, latent_stack)
